# Initial kernel scaffold; baseline (speedup 1.0000x reference)
import jax
import jax.numpy as jnp
from jax.experimental import pallas as pl


def kernel(x, edge_index, W, b):
    raise NotImplementedError("write your pallas kernel here")


# speedup vs baseline: 11.5990x; 11.5990x over previous
import functools

import jax
import jax.numpy as jnp
from jax import lax
from jax.experimental import pallas as pl
from jax.experimental.pallas import tpu as pltpu
from jax.experimental.pallas import tpu_sc as plsc

NC = 2
NS = 16
K = 128

_MESH = dict(core_axis_name="c", subcore_axis_name="s", num_cores=NC,
             num_subcores=NS)


def _deg_kernel(rows, npad):
    per_sc = rows // NC
    per_tile = rows // (NC * NS)
    nslab = npad // NS
    nchunks = nslab // K

    @functools.partial(
        pl.kernel,
        out_type=jax.ShapeDtypeStruct((NC, npad, 128), jnp.float32),
        mesh=plsc.VectorSubcoreMesh(**_MESH),
        scratch_types=[
            pltpu.VMEM((per_tile, K), jnp.int32),
            pltpu.VMEM((K, 128), jnp.float32),
            pltpu.VMEM((K, 128), jnp.float32),
            pltpu.VMEM_SHARED((npad, 128), jnp.float32),
        ],
    )
    def k(col_hbm, ones_hbm, zer_hbm, deg_out, col_v, ones_v, buf, deg_sh):
        c = lax.axis_index("c")
        s = lax.axis_index("s")
        pltpu.sync_copy(col_hbm.at[pl.ds(c * per_sc + s * per_tile, per_tile)],
                        col_v)
        pltpu.sync_copy(ones_hbm, ones_v)
        pltpu.sync_copy(zer_hbm, buf)
        for t in range(nchunks):
            pltpu.sync_copy(buf, deg_sh.at[pl.ds(s * nslab + t * K, K)])
        plsc.subcore_barrier()

        def step(j, carry):
            pltpu.sync_copy(ones_v, deg_sh.at[col_v.at[j]], add=True)
            return carry

        lax.fori_loop(0, per_tile, step, 0)
        plsc.subcore_barrier()
        for t in range(nchunks):
            pltpu.sync_copy(deg_sh.at[pl.ds(s * nslab + t * K, K)], buf)
            pltpu.sync_copy(buf, deg_out.at[c, pl.ds(s * nslab + t * K, K)])

    return k


def _scatter_kernel(rows, npad):
    per_tile = rows // NS
    nslab = npad // NS
    nchunks = nslab // K

    @functools.partial(
        pl.kernel,
        out_type=jax.ShapeDtypeStruct((NC, npad, 128), jnp.float32),
        mesh=plsc.VectorSubcoreMesh(**_MESH),
        scratch_types=[
            pltpu.VMEM((per_tile, K), jnp.int32),
            pltpu.VMEM((per_tile, K), jnp.int32),
            pltpu.VMEM((K, 128), jnp.float32),
            pltpu.VMEM_SHARED((npad, 128), jnp.float32),
        ],
    )
    def k(row_hbm, col_hbm, y_hbm, zer_hbm, s_out, row_v, col_v, buf, s_sh):
        c = lax.axis_index("c")
        s = lax.axis_index("s")
        pltpu.sync_copy(row_hbm.at[c, pl.ds(s * per_tile, per_tile)], row_v)
        pltpu.sync_copy(col_hbm.at[pl.ds(s * per_tile, per_tile)], col_v)
        pltpu.sync_copy(zer_hbm, buf)
        for t in range(nchunks):
            pltpu.sync_copy(buf, s_sh.at[pl.ds(s * nslab + t * K, K)])
        plsc.subcore_barrier()

        def step(j, carry):
            pltpu.sync_copy(y_hbm.at[row_v.at[j]], buf)
            pltpu.sync_copy(buf, s_sh.at[col_v.at[j]], add=True)
            return carry

        lax.fori_loop(0, per_tile, step, 0)
        plsc.subcore_barrier()
        for t in range(nchunks):
            pltpu.sync_copy(s_sh.at[pl.ds(s * nslab + t * K, K)], buf)
            pltpu.sync_copy(buf, s_out.at[c, pl.ds(s * nslab + t * K, K)])

    return k


def _scale_body(x_ref, d_ref, y_ref, dis_ref):
    deg = d_ref[0, :, 0:1] + d_ref[1, :, 0:1] + 1.0
    dis = lax.rsqrt(deg)
    y = x_ref[...] * dis
    y_ref[0] = y[:, :128]
    y_ref[1] = y[:, 128:]
    dis_ref[...] = jnp.broadcast_to(dis, dis_ref.shape)


def _out_body(s_ref, x_ref, dis_ref, w_ref, b_ref, o_ref):
    dis = dis_ref[:, 0:1]
    s_full = jnp.concatenate([s_ref[0], s_ref[1]], axis=1)
    z = (s_full + x_ref[...] * dis) * dis
    acc = jnp.dot(z, w_ref[...], preferred_element_type=jnp.float32)
    acc = acc + b_ref[...]
    o_ref[...] = jnp.where(acc > 0, acc, 0.1 * acc)


def kernel(x, edge_index, W, b):
    n, in_ch = x.shape
    out_ch = W.shape[1]
    e = edge_index.shape[1]
    assert in_ch == 256

    npad = -(-(n + 1) // 2048) * 2048
    rows = -(-e // (K * 32)) * 32
    epad = rows * K

    row = edge_index[0]
    col = edge_index[1]
    fill = jnp.full((epad - e,), n, dtype=jnp.int32)
    rowp = jnp.concatenate([row, fill])
    colp = jnp.concatenate([col, fill])
    col2 = colp.reshape(rows, K)
    row_both = jnp.stack([rowp, rowp + npad]).reshape(NC, rows, K)
    x_pad = jnp.pad(x, ((0, npad - n), (0, 0)))

    ones_c = jnp.ones((K, 128), jnp.float32)
    zer_k = jnp.zeros((K, 128), jnp.float32)

    deg2 = _deg_kernel(rows, npad)(col2, ones_c, zer_k)

    blk = 1024
    grid = (npad // blk,)
    y2, dis = pl.pallas_call(
        _scale_body,
        grid=grid,
        in_specs=[
            pl.BlockSpec((blk, in_ch), lambda i: (i, 0)),
            pl.BlockSpec((NC, blk, 128), lambda i: (0, i, 0)),
        ],
        out_specs=[
            pl.BlockSpec((NC, blk, 128), lambda i: (0, i, 0)),
            pl.BlockSpec((blk, 128), lambda i: (i, 0)),
        ],
        out_shape=[
            jax.ShapeDtypeStruct((NC, npad, 128), jnp.float32),
            jax.ShapeDtypeStruct((npad, 128), jnp.float32),
        ],
    )(x_pad, deg2)

    yflat = y2.reshape(NC * npad, 128)
    s_agg = _scatter_kernel(rows, npad)(row_both, col2, yflat, zer_k)

    out = pl.pallas_call(
        _out_body,
        grid=grid,
        in_specs=[
            pl.BlockSpec((NC, blk, 128), lambda i: (0, i, 0)),
            pl.BlockSpec((blk, in_ch), lambda i: (i, 0)),
            pl.BlockSpec((blk, 128), lambda i: (i, 0)),
            pl.BlockSpec((in_ch, out_ch), lambda i: (0, 0)),
            pl.BlockSpec((1, out_ch), lambda i: (0, 0)),
        ],
        out_specs=pl.BlockSpec((blk, out_ch), lambda i: (i, 0)),
        out_shape=jax.ShapeDtypeStruct((npad, out_ch), jnp.float32),
    )(s_agg, x_pad, dis, W, b.reshape(1, out_ch))

    return out[:n]
